# baseline (speedup 1.0000x reference)
import functools

import jax
import jax.numpy as jnp
from jax import lax
from jax.experimental import pallas as pl
from jax.experimental.pallas import tpu as pltpu
from jax.experimental.pallas import tpu_sc as plsc

NUM_ROWS = 8192
NUM_LBL = 32
NC = 2
NS = 16
L = 16
NW = NC * NS
RPW = NUM_ROWS // NW
NG = RPW // L
PAD = 8


def _decode_body(logits_hbm, best_hbm, keep_hbm, tok_hbm,
                 rows_v, best_v, keep_v, tok_v):
    wid = lax.axis_index("s") * NC + lax.axis_index("c")
    base = wid * RPW
    iota = lax.iota(jnp.int32, L)

    @pl.when(wid == 0)
    def _():
        pltpu.sync_copy(logits_hbm.at[pl.ds(0, RPW * NUM_LBL)],
                        rows_v.at[pl.ds(NUM_LBL, RPW * NUM_LBL)])

    @pl.when(wid != 0)
    def _():
        pltpu.sync_copy(logits_hbm.at[pl.ds((base - 1) * NUM_LBL,
                                            (RPW + 1) * NUM_LBL)], rows_v)

    def argmax16(ridx):
        rbase = ridx * NUM_LBL
        bv = plsc.load_gather(rows_v, [rbase + iota])
        bi = iota
        for l in range(1, NUM_LBL):
            colv = jnp.bitwise_and(iota + l, NUM_LBL - 1)
            v = plsc.load_gather(rows_v, [rbase + colv])
            better = (v > bv) | ((v == bv) & (colv < bi))
            bv = jnp.where(better, v, bv)
            bi = jnp.where(better, colv, bi)
        return bi

    bi_m1 = argmax16(iota)
    bi_m1 = jnp.where(jnp.broadcast_to(wid == 0, (L,)),
                      jnp.full((L,), -1, jnp.int32), bi_m1)
    plsc.store_scatter(best_v, [jnp.full((L,), PAD - 1, jnp.int32)],
                       bi_m1, mask=iota == 0)

    def body(g, carry):
        ridx = 1 + g * L + iota
        bi = argmax16(ridx)
        plsc.store_scatter(best_v, [PAD + g * L + iota], bi)
        prev = plsc.load_gather(best_v, [PAD - 1 + g * L + iota])
        keep = (bi != prev) & (bi >= 2)
        plsc.store_scatter(keep_v, [g * L + iota], keep.astype(jnp.int32))
        plsc.store_scatter(tok_v, [g * L + iota],
                           jnp.where(keep, bi, jnp.full((L,), -1, jnp.int32)))
        return carry

    lax.fori_loop(0, NG, body, 0)

    pltpu.sync_copy(best_v.at[pl.ds(PAD, RPW)], best_hbm.at[pl.ds(base, RPW)])
    pltpu.sync_copy(keep_v, keep_hbm.at[pl.ds(base, RPW)])
    pltpu.sync_copy(tok_v, tok_hbm.at[pl.ds(base, RPW)])


@functools.cache
def _build_decode():
    return functools.partial(
        pl.kernel,
        out_type=(jax.ShapeDtypeStruct((NUM_ROWS,), jnp.int32),) * 3,
        mesh=plsc.VectorSubcoreMesh(core_axis_name="c", subcore_axis_name="s",
                                    num_cores=NC, num_subcores=NS),
        compiler_params=pltpu.CompilerParams(needs_layout_passes=False),
        scratch_types=[
            pltpu.VMEM(((RPW + 1) * NUM_LBL,), jnp.float32),
            pltpu.VMEM((RPW + PAD,), jnp.int32),
            pltpu.VMEM((RPW,), jnp.int32),
            pltpu.VMEM((RPW,), jnp.int32),
        ],
    )(_decode_body)


def kernel(logits):
    best, keep, tok = _build_decode()(logits.reshape(-1))
    return best, keep.astype(bool), tok


# speedup vs baseline: 1.0118x; 1.0118x over previous
import functools

import jax
import jax.numpy as jnp
from jax import lax
from jax.experimental import pallas as pl
from jax.experimental.pallas import tpu as pltpu
from jax.experimental.pallas import tpu_sc as plsc

NUM_ROWS = 8192
NUM_LBL = 32
NC = 2
NS = 16
L = 16
NW = NC * NS
RPW = NUM_ROWS // NW
NG = RPW // L + 1
PRE = 8
BIG = 3.0e38


def _decode_body(logits_hbm, best_hbm, keep_hbm, tok_hbm,
                 rows_v, best_v, keep_v, tok_v):
    wid = lax.axis_index("s") * NC + lax.axis_index("c")
    base = wid * RPW
    iota = lax.iota(jnp.int32, L)

    pltpu.sync_copy(logits_hbm.at[pl.ds(base, RPW)], rows_v.at[pl.ds(PRE, RPW)])

    @pl.when(wid != 0)
    def _():
        pltpu.sync_copy(logits_hbm.at[pl.ds(base - PRE, PRE)],
                        rows_v.at[pl.ds(0, PRE)])

    @pl.when(wid == 0)
    def _():
        row7 = jnp.full((L,), PRE - 1, jnp.int32)
        plsc.store_scatter(rows_v, [row7, iota],
                           jnp.where(iota == 0, BIG, -BIG))
        plsc.store_scatter(rows_v, [row7, iota + L], jnp.full((L,), -BIG))

    def body(g, carry):
        ridx = g * L + iota
        bv = plsc.load_gather(rows_v, [ridx, iota])
        bi = iota
        for l in range(1, NUM_LBL):
            colv = jnp.bitwise_and(iota + l, NUM_LBL - 1)
            v = plsc.load_gather(rows_v, [ridx, colv])
            better = (v > bv) | ((v == bv) & (colv < bi))
            bv = jnp.where(better, v, bv)
            bi = jnp.where(better, colv, bi)
        best_v[pl.ds(PRE + g * L, L)] = bi
        prev = plsc.load_gather(best_v, [PRE - 1 + ridx])
        keep = (bi != prev) & (bi >= 2)
        keep_v[pl.ds(g * L, L)] = keep.astype(jnp.int32)
        tok_v[pl.ds(g * L, L)] = jnp.where(keep, bi,
                                           jnp.full((L,), -1, jnp.int32))
        return carry

    lax.fori_loop(0, NG, body, 0)

    pltpu.sync_copy(best_v.at[pl.ds(2 * PRE, RPW)],
                    best_hbm.at[pl.ds(base, RPW)])
    pltpu.sync_copy(keep_v.at[pl.ds(PRE, RPW)], keep_hbm.at[pl.ds(base, RPW)])
    pltpu.sync_copy(tok_v.at[pl.ds(PRE, RPW)], tok_hbm.at[pl.ds(base, RPW)])


@functools.cache
def _build_decode():
    return functools.partial(
        pl.kernel,
        out_type=(jax.ShapeDtypeStruct((NUM_ROWS,), jnp.int32),) * 3,
        mesh=plsc.VectorSubcoreMesh(core_axis_name="c", subcore_axis_name="s",
                                    num_cores=NC, num_subcores=NS),
        compiler_params=pltpu.CompilerParams(needs_layout_passes=False,
                                             use_tc_tiling_on_sc=False),
        scratch_types=[
            pltpu.VMEM((NG * L + PRE, NUM_LBL), jnp.float32),
            pltpu.VMEM((NG * L + 2 * PRE,), jnp.int32),
            pltpu.VMEM((NG * L,), jnp.int32),
            pltpu.VMEM((NG * L,), jnp.int32),
        ],
    )(_decode_body)


def kernel(logits):
    best, keep, tok = _build_decode()(logits)
    return best, keep.astype(bool), tok
